# Initial kernel scaffold; baseline (speedup 1.0000x reference)
import jax
import jax.numpy as jnp
from jax.experimental import pallas as pl


def kernel(X, A, A2, W0, a0, W1, a1, d0_w, d0_b, d1_w, d1_b, out_w, out_b):
    raise NotImplementedError("write your pallas kernel here")


# speedup vs baseline: 1.7101x; 1.7101x over previous
import functools

import jax
import jax.numpy as jnp
from jax.experimental import pallas as pl
from jax.experimental.pallas import tpu as pltpu

N = 4096
DA = 64
BLK = 256
NEG = -9e15


def _attn_body(x_ref, a_ref, w_ref, av_ref, o_ref, wh_ref, dt_ref):
    b = pl.program_id(0)

    @pl.when(b == 0)
    def _():
        wh = jnp.dot(x_ref[...], w_ref[...], preferred_element_type=jnp.float32)
        wh_ref[...] = wh
        dt_ref[...] = jax.lax.dot_general(
            av_ref[...][DA:, :], wh, (((0,), (1,)), ((), ())),
            preferred_element_type=jnp.float32)

    wh_blk = wh_ref[pl.ds(b * BLK, BLK), :]
    s = jnp.dot(wh_blk, av_ref[...][:DA, :], preferred_element_type=jnp.float32)
    e = s + dt_ref[...]
    e = jnp.where(e > 0, e, 0.2 * e)
    e = jnp.where(a_ref[...] > 0, e, NEG)
    m = jnp.max(e, axis=1, keepdims=True)
    p = jnp.exp(e - m)
    denom = jnp.sum(p, axis=1, keepdims=True)
    acc = jnp.dot(p, wh_ref[...], preferred_element_type=jnp.float32) / denom
    o_ref[...] = jnp.where(acc > 0, acc, jnp.exp(acc) - 1.0)


def _attn(x, a_mask, w, av):
    n_feat = x.shape[1]
    return pl.pallas_call(
        _attn_body,
        grid=(N // BLK,),
        in_specs=[
            pl.BlockSpec((N, n_feat), lambda b: (0, 0)),
            pl.BlockSpec((BLK, N), lambda b: (b, 0)),
            pl.BlockSpec((n_feat, DA), lambda b: (0, 0)),
            pl.BlockSpec((2 * DA, 1), lambda b: (0, 0)),
        ],
        out_specs=pl.BlockSpec((BLK, DA), lambda b: (b, 0)),
        out_shape=jax.ShapeDtypeStruct((N, DA), jnp.float32),
        scratch_shapes=[
            pltpu.VMEM((N, DA), jnp.float32),
            pltpu.VMEM((1, N), jnp.float32),
        ],
    )(x, a_mask, w, av)


def _head_body(x1_ref, x2_ref, w0_ref, b0_ref, w1_ref, b1_ref, w2_ref, b2_ref,
               o_ref):
    xg = jnp.sum(x2_ref[...] - x1_ref[...], axis=0, keepdims=True)
    xg = xg * jnp.float32(1.0 / N)
    h = jnp.dot(xg, w0_ref[...], preferred_element_type=jnp.float32) + b0_ref[...]
    h = jnp.maximum(h, 0.0)
    h = jnp.dot(h, w1_ref[...], preferred_element_type=jnp.float32) + b1_ref[...]
    h = jnp.maximum(h, 0.0)
    z = jnp.dot(h, w2_ref[...], preferred_element_type=jnp.float32) + b2_ref[...]
    z = z - jnp.max(z, axis=1, keepdims=True)
    p = jnp.exp(z)
    o_ref[...] = p / jnp.sum(p, axis=1, keepdims=True)


def _head(x1, x2, d0_w, d0_b, d1_w, d1_b, out_w, out_b):
    return pl.pallas_call(
        _head_body,
        out_shape=jax.ShapeDtypeStruct((1, out_w.shape[1]), jnp.float32),
    )(x1, x2, d0_w, d0_b, d1_w, d1_b, out_w, out_b)


def kernel(X, A, A2, W0, a0, W1, a1, d0_w, d0_b, d1_w, d1_b, out_w, out_b):
    x1 = _attn(X, A, W0, a0)
    x2 = _attn(X, A2, W0, a0)
    x1 = _attn(x1, A, W1, a1)
    x2 = _attn(x2, A2, W1, a1)
    return _head(x1, x2, d0_w, d0_b.reshape(1, -1), d1_w, d1_b.reshape(1, -1),
                 out_w, out_b.reshape(1, -1))


# speedup vs baseline: 1.9908x; 1.1641x over previous
import jax
import jax.numpy as jnp
from jax.experimental import pallas as pl
from jax.experimental.pallas import tpu as pltpu

N = 4096
DA = 64
BLK = 256
LOG2E = 1.4426950408889634


def _attn_body(x_ref, a_ref, w_ref, av_ref, o_ref,
               wh_ref, whb_ref, dt_ref, dt2_ref, fb_ref):
    b = pl.program_id(0)

    @pl.when(b == 0)
    def _():
        wh = jnp.dot(x_ref[...], w_ref[...], preferred_element_type=jnp.float32)
        wh_ref[...] = wh
        whb_ref[...] = wh.astype(jnp.bfloat16)
        dt = jax.lax.dot_general(
            av_ref[...][DA:, :], wh, (((0,), (1,)), ((), ())),
            preferred_element_type=jnp.float32) * LOG2E
        dt_ref[...] = dt
        dt2_ref[...] = 0.2 * dt
        cm = jnp.sum(wh, axis=0, keepdims=True) * (1.0 / N)
        fb_ref[...] = jnp.where(cm > 0, cm, jnp.exp(cm) - 1.0)

    wh_blk = wh_ref[pl.ds(b * BLK, BLK), :]
    st = jnp.dot(wh_blk, av_ref[...][:DA, :],
                 preferred_element_type=jnp.float32) * LOG2E
    dmax = jnp.max(dt_ref[...], axis=1, keepdims=True)
    t = st + dmax
    mt = jnp.maximum(t, 0.2 * t)
    s1 = st - mt
    s2 = 0.2 * st - mt
    u = s1 + dt_ref[...]
    v = s2 + dt2_ref[...]
    p = jnp.exp2(jnp.maximum(u, v)) * a_ref[...]
    denom = jnp.sum(p, axis=1, keepdims=True)
    acc = jnp.dot(p.astype(jnp.bfloat16), whb_ref[...],
                  preferred_element_type=jnp.float32)
    acc = acc * jnp.where(denom > 0, 1.0 / denom, 0.0)
    acc = jnp.where(acc > 0, acc, jnp.exp(acc) - 1.0)
    o_ref[...] = jnp.where(denom > 0, acc, fb_ref[...])


def _attn(x, a_mask, w, av):
    n_feat = x.shape[1]
    return pl.pallas_call(
        _attn_body,
        grid=(N // BLK,),
        in_specs=[
            pl.BlockSpec((N, n_feat), lambda b: (0, 0)),
            pl.BlockSpec((BLK, N), lambda b: (b, 0)),
            pl.BlockSpec((n_feat, DA), lambda b: (0, 0)),
            pl.BlockSpec((2 * DA, 1), lambda b: (0, 0)),
        ],
        out_specs=pl.BlockSpec((BLK, DA), lambda b: (b, 0)),
        out_shape=jax.ShapeDtypeStruct((N, DA), jnp.float32),
        scratch_shapes=[
            pltpu.VMEM((N, DA), jnp.float32),
            pltpu.VMEM((N, DA), jnp.bfloat16),
            pltpu.VMEM((1, N), jnp.float32),
            pltpu.VMEM((1, N), jnp.float32),
            pltpu.VMEM((1, DA), jnp.float32),
        ],
    )(x, a_mask, w, av)


def _head_body(x1_ref, x2_ref, w0_ref, b0_ref, w1_ref, b1_ref, w2_ref, b2_ref,
               o_ref):
    xg = jnp.sum(x2_ref[...] - x1_ref[...], axis=0, keepdims=True)
    xg = xg * jnp.float32(1.0 / N)
    h = jnp.dot(xg, w0_ref[...], preferred_element_type=jnp.float32) + b0_ref[...]
    h = jnp.maximum(h, 0.0)
    h = jnp.dot(h, w1_ref[...], preferred_element_type=jnp.float32) + b1_ref[...]
    h = jnp.maximum(h, 0.0)
    z = jnp.dot(h, w2_ref[...], preferred_element_type=jnp.float32) + b2_ref[...]
    z = z - jnp.max(z, axis=1, keepdims=True)
    p = jnp.exp(z)
    o_ref[...] = p / jnp.sum(p, axis=1, keepdims=True)


def _head(x1, x2, d0_w, d0_b, d1_w, d1_b, out_w, out_b):
    return pl.pallas_call(
        _head_body,
        out_shape=jax.ShapeDtypeStruct((1, out_w.shape[1]), jnp.float32),
    )(x1, x2, d0_w, d0_b, d1_w, d1_b, out_w, out_b)


def kernel(X, A, A2, W0, a0, W1, a1, d0_w, d0_b, d1_w, d1_b, out_w, out_b):
    x1 = _attn(X, A, W0, a0)
    x2 = _attn(X, A2, W0, a0)
    x1 = _attn(x1, A, W1, a1)
    x2 = _attn(x2, A2, W1, a1)
    return _head(x1, x2, d0_w, d0_b.reshape(1, -1), d1_w, d1_b.reshape(1, -1),
                 out_w, out_b.reshape(1, -1))


# speedup vs baseline: 2.2643x; 1.1374x over previous
import jax
import jax.numpy as jnp
from jax.experimental import pallas as pl
from jax.experimental.pallas import tpu as pltpu

N = 4096
DA = 64
BLK = 512
LOG2E = 1.4426950408889634


def _attn_body(x_ref, a_ref, w_ref, av_ref, o_ref,
               wh_ref, whb_ref, dt_ref, dt2_ref, fb_ref):
    b = pl.program_id(0)

    @pl.when(b == 0)
    def _():
        wh = jnp.dot(x_ref[...], w_ref[...], preferred_element_type=jnp.float32)
        wh_ref[...] = wh
        whb_ref[...] = wh.astype(jnp.bfloat16)
        dt = jax.lax.dot_general(
            av_ref[...][DA:, :], wh, (((0,), (1,)), ((), ())),
            preferred_element_type=jnp.float32) * LOG2E
        dt_ref[...] = dt
        dt2_ref[...] = 0.2 * dt
        cm = jnp.sum(wh, axis=0, keepdims=True) * (1.0 / N)
        fb_ref[...] = jnp.where(cm > 0, cm, jnp.exp(cm) - 1.0)

    wh_blk = wh_ref[pl.ds(b * BLK, BLK), :]
    st = jnp.dot(wh_blk, av_ref[...][:DA, :],
                 preferred_element_type=jnp.float32) * LOG2E
    dmax = jnp.max(dt_ref[...], axis=1, keepdims=True)
    t = st + dmax
    mt = jnp.maximum(t, 0.2 * t)
    s1 = st - mt
    s2 = 0.2 * st - mt
    u = s1 + dt_ref[...]
    v = s2 + dt2_ref[...]
    p = jnp.exp2(jnp.maximum(u, v)) * a_ref[...]
    denom = jnp.sum(p, axis=1, keepdims=True)
    acc = jnp.dot(p.astype(jnp.bfloat16), whb_ref[...],
                  preferred_element_type=jnp.float32)
    acc = acc * jnp.where(denom > 0, 1.0 / denom, 0.0)
    acc = jnp.where(acc > 0, acc, jnp.exp(acc) - 1.0)
    o_ref[...] = jnp.where(denom > 0, acc, fb_ref[...])


def _attn(x, a_mask, w, av):
    n_feat = x.shape[1]
    return pl.pallas_call(
        _attn_body,
        grid=(N // BLK,),
        in_specs=[
            pl.BlockSpec((N, n_feat), lambda b: (0, 0)),
            pl.BlockSpec((BLK, N), lambda b: (b, 0)),
            pl.BlockSpec((n_feat, DA), lambda b: (0, 0)),
            pl.BlockSpec((2 * DA, 1), lambda b: (0, 0)),
        ],
        out_specs=pl.BlockSpec((BLK, DA), lambda b: (b, 0)),
        out_shape=jax.ShapeDtypeStruct((N, DA), jnp.float32),
        scratch_shapes=[
            pltpu.VMEM((N, DA), jnp.float32),
            pltpu.VMEM((N, DA), jnp.bfloat16),
            pltpu.VMEM((1, N), jnp.float32),
            pltpu.VMEM((1, N), jnp.float32),
            pltpu.VMEM((1, DA), jnp.float32),
        ],
    )(x, a_mask, w, av)


def _head_body(x1_ref, x2_ref, w0_ref, b0_ref, w1_ref, b1_ref, w2_ref, b2_ref,
               o_ref):
    xg = jnp.sum(x2_ref[...] - x1_ref[...], axis=0, keepdims=True)
    xg = xg * jnp.float32(1.0 / N)
    h = jnp.dot(xg, w0_ref[...], preferred_element_type=jnp.float32) + b0_ref[...]
    h = jnp.maximum(h, 0.0)
    h = jnp.dot(h, w1_ref[...], preferred_element_type=jnp.float32) + b1_ref[...]
    h = jnp.maximum(h, 0.0)
    z = jnp.dot(h, w2_ref[...], preferred_element_type=jnp.float32) + b2_ref[...]
    z = z - jnp.max(z, axis=1, keepdims=True)
    p = jnp.exp(z)
    o_ref[...] = p / jnp.sum(p, axis=1, keepdims=True)


def _head(x1, x2, d0_w, d0_b, d1_w, d1_b, out_w, out_b):
    return pl.pallas_call(
        _head_body,
        out_shape=jax.ShapeDtypeStruct((1, out_w.shape[1]), jnp.float32),
    )(x1, x2, d0_w, d0_b, d1_w, d1_b, out_w, out_b)


def kernel(X, A, A2, W0, a0, W1, a1, d0_w, d0_b, d1_w, d1_b, out_w, out_b):
    x1 = _attn(X, A, W0, a0)
    x2 = _attn(X, A2, W0, a0)
    x1 = _attn(x1, A, W1, a1)
    x2 = _attn(x2, A2, W1, a1)
    return _head(x1, x2, d0_w, d0_b.reshape(1, -1), d1_w, d1_b.reshape(1, -1),
                 out_w, out_b.reshape(1, -1))
